# baseline (speedup 1.0000x reference)
import functools

import jax
import jax.numpy as jnp
from jax import lax
from jax.experimental import pallas as pl
from jax.experimental.pallas import tpu as pltpu
from jax.experimental.pallas import tpu_sc as plsc

_FDIM = 8
_LODS = (32, 64, 128)
_N = 524288
_NC, _NS = 2, 16
_NW = _NC * _NS
_P = 256
_PW = _N // _NW
_NCHUNK = _PW // _P
_G = _P // 16
_JG = (4 * _P) // 128
_NCB = _P // 128


def _scaled_coords(v, res):
    p = v / 3.0 + 0.5
    s = p * float(res - 1)
    i = jnp.minimum(jnp.maximum(s, 0.0).astype(jnp.int32), res - 2)
    f = s - i.astype(jnp.float32)
    return i, f


def _interp_sc_body(ptsT, cb0, cb1, cb2, feats,
                    xyzA, idx0A, idx1A, idx2A, rows0A, rows1A, rows2A,
                    xyzB, idx0B, idx1B, idx2B, rows0B, rows1B, rows2B,
                    outv, sem):
    wid = lax.axis_index("c") * _NS + lax.axis_index("s")
    cbs = (cb0, cb1, cb2)
    bufA = (xyzA, (idx0A, idx1A, idx2A), (rows0A, rows1A, rows2A))
    bufB = (xyzB, (idx0B, idx1B, idx2B), (rows0B, rows1B, rows2B))
    iota = lax.iota(jnp.int32, 16)

    def fire(k, buf):
        xyz, idxs, rows = buf
        base = wid * _PW + k * _P
        pltpu.sync_copy(ptsT.at[:, pl.ds(base, _P)], xyz)

        def group_a(g, c2):
            x = xyz[0, pl.ds(g * 16, 16)]
            y = xyz[1, pl.ds(g * 16, 16)]
            z = xyz[2, pl.ds(g * 16, 16)]
            for l, res in enumerate(_LODS):
                ix, _ = _scaled_coords(x, res)
                iy, _ = _scaled_coords(y, res)
                iz, _ = _scaled_coords(z, res)
                flat = (ix * res + iy) * res + iz
                for c in range(4):
                    dx, dy = (c >> 1) & 1, c & 1
                    off = dx * res * res + dy * res
                    idxs[l][pl.ds(c * _P + g * 16, 16)] = flat + off
            return c2

        lax.fori_loop(0, _G, group_a, 0)
        for l in range(3):
            for j in range(_JG):
                pltpu.async_copy(
                    cbs[l].at[idxs[l].at[pl.ds(j * 128, 128)]],
                    rows[l].at[pl.ds(j * 128, 128)], sem)

    def drain(buf):
        _, _, rows = buf
        for l in range(3):
            pltpu.make_async_copy(
                cbs[l].at[pl.ds(0, 4 * _P)], rows[l], sem).wait()

    def combine(k, buf):
        xyz, _, rows = buf

        def group_b(g, c2):
            x = xyz[0, pl.ds(g * 16, 16)]
            y = xyz[1, pl.ds(g * 16, 16)]
            z = xyz[2, pl.ds(g * 16, 16)]
            rowbase = g * 16 + iota
            for l, res in enumerate(_LODS):
                _, fx = _scaled_coords(x, res)
                _, fy = _scaled_coords(y, res)
                _, fz = _scaled_coords(z, res)
                wx = (1.0 - fx, fx)
                wy = (1.0 - fy, fy)
                wz = (1.0 - fz, fz)
                acc = [jnp.zeros((16,), jnp.float32) for _ in range(_FDIM)]
                for c in range(4):
                    dx, dy = (c >> 1) & 1, c & 1
                    wxy = wx[dx] * wy[dy]
                    r = rowbase + c * _P
                    for dz in range(2):
                        w = wxy * wz[dz]
                        for f in range(_FDIM):
                            v = plsc.load_gather(
                                rows[l],
                                [r, jnp.full((16,), dz * _FDIM + f,
                                             jnp.int32)])
                            acc[f] = acc[f] + w * v
                cbi = g // 8
                lo = (g % 8) * 16
                for f in range(_FDIM):
                    outv[l, cbi, f, pl.ds(lo, 16)] = acc[f]
            return c2

        lax.fori_loop(0, _G, group_b, 0)
        pltpu.sync_copy(outv, feats.at[:, pl.ds(wid * (_PW // 128) + k * _NCB,
                                                _NCB), :, :])

    fire(0, bufA)

    def pair_body(j, carry):
        fire(2 * j + 1, bufB)
        drain(bufA)
        combine(2 * j, bufA)
        fire(2 * j + 2, bufA)
        drain(bufB)
        combine(2 * j + 1, bufB)
        return carry

    lax.fori_loop(0, _NCHUNK // 2 - 1, pair_body, 0)
    fire(_NCHUNK - 1, bufB)
    drain(bufA)
    combine(_NCHUNK - 2, bufA)
    drain(bufB)
    combine(_NCHUNK - 1, bufB)


_TBLK = (32768 // 128, 262144 // 128, 2097152 // 128)
_TSPAN = 8


def _transpose_sc_body(cbt0, cbt1, cbt2, out0, out1, out2,
                       bufA, bufB, obufA, obufB, insem, outsem):
    wid = lax.axis_index("c") * _NS + lax.axis_index("s")
    iota = lax.iota(jnp.int32, 16)
    fcol = [jnp.full((16,), f, jnp.int32) for f in range(2 * _FDIM)]

    for cbt, out, nblk in ((cbt0, out0, _TBLK[0]), (cbt1, out1, _TBLK[1]),
                           (cbt2, out2, _TBLK[2])):
        per_w = nblk // _NW
        base_blk = wid * per_w
        spans = per_w // _TSPAN

        def infire(s, buf, cbt=cbt, nblk=nblk):
            blk0 = base_blk + s * _TSPAN
            pltpu.async_copy(cbt.at[pl.ds(blk0, _TSPAN), :, :],
                             buf.at[pl.ds(0, _TSPAN), :, :], insem)
            nxt = jnp.minimum(blk0 + _TSPAN, nblk - 1)
            pltpu.async_copy(cbt.at[pl.ds(nxt, 1), :, :],
                             buf.at[pl.ds(_TSPAN, 1), :, :], insem)

        def inwait(buf, cbt=cbt):
            pltpu.make_async_copy(
                cbt.at[pl.ds(0, _TSPAN + 1), :, :], buf, insem).wait()

        def shuffle(buf, obuf):
            def blk_body(b, c2):
                for g in range(8):
                    rowi = b * 128 + g * 16 + iota
                    ip = b * 128 + g * 16 + 1 + iota
                    blkv = lax.shift_right_logical(ip, 7)
                    lanev = lax.bitwise_and(ip, 127)
                    for f in range(_FDIM):
                        v = buf[b, f, pl.ds(g * 16, 16)]
                        plsc.store_scatter(obuf, [rowi, fcol[f]], v)
                        vp = plsc.load_gather(buf, [blkv, fcol[f], lanev])
                        plsc.store_scatter(obuf, [rowi, fcol[_FDIM + f]], vp)
                return c2

            lax.fori_loop(0, _TSPAN, blk_body, 0)

        def outfire(s, obuf, out=out):
            blk0 = base_blk + s * _TSPAN
            pltpu.async_copy(obuf, out.at[pl.ds(blk0 * 128, _TSPAN * 128), :],
                             outsem)

        def outwait(obuf, out=out):
            pltpu.make_async_copy(
                obuf, out.at[pl.ds(0, _TSPAN * 128), :], outsem).wait()

        if spans == 1:
            infire(0, bufA)
            inwait(bufA)
            shuffle(bufA, obufA)
            outfire(0, obufA)
            outwait(obufA)
        else:
            infire(0, bufA)
            infire(1, bufB)
            inwait(bufA)
            shuffle(bufA, obufA)
            outfire(0, obufA)
            infire(2, bufA)
            inwait(bufB)
            shuffle(bufB, obufB)
            outfire(1, obufB)

            def pair_body(j, c2):
                infire(2 * j + 1, bufB)
                inwait(bufA)
                outwait(obufA)
                shuffle(bufA, obufA)
                outfire(2 * j, obufA)
                infire(jnp.minimum(2 * j + 2, spans - 1), bufA)
                inwait(bufB)
                outwait(obufB)
                shuffle(bufB, obufB)
                outfire(2 * j + 1, obufB)
                return c2

            lax.fori_loop(1, spans // 2, pair_body, 0)
            inwait(bufA)
            outwait(obufA)
            outwait(obufB)


_transpose_sc_cache = []


def _transpose_sc(cbt0, cbt1, cbt2):
    if not _transpose_sc_cache:
        _transpose_sc_cache.append(functools.partial(
            pl.kernel,
            out_type=(
                jax.ShapeDtypeStruct((32768, 2 * _FDIM), jnp.float32),
                jax.ShapeDtypeStruct((262144, 2 * _FDIM), jnp.float32),
                jax.ShapeDtypeStruct((2097152, 2 * _FDIM), jnp.float32),
            ),
            mesh=plsc.VectorSubcoreMesh(
                core_axis_name="c", subcore_axis_name="s",
                num_cores=_NC, num_subcores=_NS),
            scratch_types=[
                pltpu.VMEM((_TSPAN + 1, _FDIM, 128), jnp.float32),
                pltpu.VMEM((_TSPAN + 1, _FDIM, 128), jnp.float32),
                pltpu.VMEM((_TSPAN * 128, 2 * _FDIM), jnp.float32),
                pltpu.VMEM((_TSPAN * 128, 2 * _FDIM), jnp.float32),
                pltpu.SemaphoreType.DMA,
                pltpu.SemaphoreType.DMA,
            ],
            compiler_params=pltpu.CompilerParams(
                needs_layout_passes=False, use_tc_tiling_on_sc=False),
        )(_transpose_sc_body))
    return _transpose_sc_cache[0](cbt0, cbt1, cbt2)


_interp_sc_cache = []


def _interp_sc(*args):
    if not _interp_sc_cache:
        _interp_sc_cache.append(functools.partial(
            pl.kernel,
            out_type=jax.ShapeDtypeStruct((3, _N // 128, _FDIM, 128),
                                          jnp.float32),
            mesh=plsc.VectorSubcoreMesh(
                core_axis_name="c", subcore_axis_name="s",
                num_cores=_NC, num_subcores=_NS),
            scratch_types=(
                [pltpu.VMEM((3, _P), jnp.float32)]
                + [pltpu.VMEM((4 * _P,), jnp.int32)] * 3
                + [pltpu.VMEM((4 * _P, 2 * _FDIM), jnp.float32)] * 3
            ) * 2 + [
                pltpu.VMEM((3, _NCB, _FDIM, 128), jnp.float32),
                pltpu.SemaphoreType.DMA,
            ],
            compiler_params=pltpu.CompilerParams(
                needs_layout_passes=False, use_tc_tiling_on_sc=False),
        )(_interp_sc_body))
    return _interp_sc_cache[0](*args)


_BT = 4096


def _mm(w, x):
    return lax.dot_general(
        w.astype(jnp.bfloat16), x.astype(jnp.bfloat16),
        (((1,), (0,)), ((), ())), preferred_element_type=jnp.float32)


def _mlp_body(feats_ref, ptsT_ref, dT_ref, sw1t, sb1, sw2t, sb2,
              cw1ht, cw1et, cb1, cw2t, cb2, cw3t, cb3,
              colorT_ref, sigmaT_ref):
    f = feats_ref[...]
    pts = ptsT_ref[...]
    dd = dT_ref[...]
    p = pts / 3.0
    m = ((jnp.abs(p[0:1, :]) < 0.5) & (jnp.abs(p[1:2, :]) < 0.5)
         & (jnp.abs(p[2:3, :]) < 0.5))
    s = jnp.sin(dd)
    c = jnp.cos(dd)
    embs = [dd, s, c]
    for _ in range(3):
        s, c = 2.0 * s * c, 1.0 - 2.0 * s * s
        embs.append(s)
        embs.append(c)
    emb = jnp.concatenate(embs, axis=0)
    h1 = jnp.maximum(_mm(sw1t[...], f) + sb1[...], 0.0)
    h = _mm(sw2t[...], h1) + sb2[...]
    x1 = jnp.maximum(_mm(cw1ht[...], h) + _mm(cw1et[...], emb) + cb1[...], 0.0)
    x2 = jnp.maximum(_mm(cw2t[...], x1) + cb2[...], 0.0)
    logits = _mm(cw3t[...], x2) + cb3[...]
    cl = 1.0 / (1.0 + jnp.exp(-logits))
    colorT_ref[...] = jnp.where(m, cl, 0.0)
    ls = jnp.where(m, h[0:1, :], -100000.0)
    sigmaT_ref[...] = jnp.exp(ls)


def _full_spec(shape):
    return pl.BlockSpec(shape, lambda i: (0,) * len(shape))


def _mlp_call(feats, ptsT, dT, wts):
    grid = (_N // _BT,)
    in_specs = [
        pl.BlockSpec((3 * _FDIM, _BT), lambda i: (0, i)),
        pl.BlockSpec((3, _BT), lambda i: (0, i)),
        pl.BlockSpec((3, _BT), lambda i: (0, i)),
    ] + [_full_spec(w.shape) for w in wts]
    out_specs = [
        pl.BlockSpec((3, _BT), lambda i: (0, i)),
        pl.BlockSpec((1, _BT), lambda i: (0, i)),
    ]
    out_shape = [
        jax.ShapeDtypeStruct((3, _N), jnp.float32),
        jax.ShapeDtypeStruct((1, _N), jnp.float32),
    ]
    return pl.pallas_call(
        _mlp_body, grid=grid, in_specs=in_specs, out_specs=out_specs,
        out_shape=out_shape)(feats, ptsT, dT, *wts)


def kernel(pts, d, codebook0, codebook1, codebook2, sw1, sb1, sw2, sb2,
           cw1, cb1, cw2, cb2, cw3, cb3):
    ptsT = pts.T
    dT = d.T
    cbt0 = codebook0.T.reshape(_FDIM, -1, 128).transpose(1, 0, 2)
    cbt1 = codebook1.T.reshape(_FDIM, -1, 128).transpose(1, 0, 2)
    cbt2 = codebook2.T.reshape(_FDIM, -1, 128).transpose(1, 0, 2)
    cb0l, cb1l, cb2l = _transpose_sc(cbt0, cbt1, cbt2)
    feats4 = _interp_sc(ptsT, cb0l, cb1l, cb2l)
    feats = feats4.transpose(0, 2, 1, 3).reshape(3 * _FDIM, _N)
    wts = (
        sw1.T, sb1.reshape(64, 1),
        sw2.T, sb2.reshape(16, 1),
        cw1[:16].T, cw1[16:].T, cb1.reshape(64, 1),
        cw2.T, cb2.reshape(64, 1),
        cw3.T, cb3.reshape(3, 1),
    )
    colorT, sigmaT = _mlp_call(feats, ptsT, dT, wts)
    return colorT.T, sigmaT.T


# speedup vs baseline: 1.0309x; 1.0309x over previous
import functools

import jax
import jax.numpy as jnp
from jax import lax
from jax.experimental import pallas as pl
from jax.experimental.pallas import tpu as pltpu
from jax.experimental.pallas import tpu_sc as plsc

_FDIM = 8
_LODS = (32, 64, 128)
_N = 524288
_NC, _NS = 2, 16
_NW = _NC * _NS
_P = 256
_PW = _N // _NW
_NCHUNK = _PW // _P
_G = _P // 16
_JG = (4 * _P) // 128
_NCB = _P // 128


def _scaled_coords(v, res):
    p = v / 3.0 + 0.5
    s = p * float(res - 1)
    i = jnp.minimum(jnp.maximum(s, 0.0).astype(jnp.int32), res - 2)
    f = s - i.astype(jnp.float32)
    return i, f


def _interp_sc_body(ptsT, cb0, cb1, cb2, feats,
                    xyzA, idx0A, idx1A, idx2A, rows0A, rows1A, rows2A,
                    xyzB, idx0B, idx1B, idx2B, rows0B, rows1B, rows2B,
                    outv, sem):
    wid = lax.axis_index("c") * _NS + lax.axis_index("s")
    cbs = (cb0, cb1, cb2)
    bufA = (xyzA, (idx0A, idx1A, idx2A), (rows0A, rows1A, rows2A))
    bufB = (xyzB, (idx0B, idx1B, idx2B), (rows0B, rows1B, rows2B))
    iota = lax.iota(jnp.int32, 16)

    def fire(k, buf):
        xyz, idxs, rows = buf
        base = wid * _PW + k * _P
        pltpu.sync_copy(ptsT.at[:, pl.ds(base, _P)], xyz)

        @plsc.parallel_loop(0, _G)
        def group_a(g):
            x = xyz[0, pl.ds(g * 16, 16)]
            y = xyz[1, pl.ds(g * 16, 16)]
            z = xyz[2, pl.ds(g * 16, 16)]
            for l, res in enumerate(_LODS):
                ix, _ = _scaled_coords(x, res)
                iy, _ = _scaled_coords(y, res)
                iz, _ = _scaled_coords(z, res)
                flat = (ix * res + iy) * res + iz
                for c in range(4):
                    dx, dy = (c >> 1) & 1, c & 1
                    off = dx * res * res + dy * res
                    idxs[l][pl.ds(c * _P + g * 16, 16)] = flat + off

        for l in range(3):
            for j in range(_JG):
                pltpu.async_copy(
                    cbs[l].at[idxs[l].at[pl.ds(j * 128, 128)]],
                    rows[l].at[pl.ds(j * 128, 128)], sem)

    def combine(k, buf):
        xyz, _, rows = buf
        for l, res in enumerate(_LODS):
            pltpu.make_async_copy(
                cbs[l].at[pl.ds(0, 4 * _P)], rows[l], sem).wait()

            @plsc.parallel_loop(0, _G)
            def group_b(g, l=l, res=res, rowsl=rows[l]):
                x = xyz[0, pl.ds(g * 16, 16)]
                y = xyz[1, pl.ds(g * 16, 16)]
                z = xyz[2, pl.ds(g * 16, 16)]
                rowbase = g * 16 + iota
                _, fx = _scaled_coords(x, res)
                _, fy = _scaled_coords(y, res)
                _, fz = _scaled_coords(z, res)
                wx = (1.0 - fx, fx)
                wy = (1.0 - fy, fy)
                wz = (1.0 - fz, fz)
                acc = [jnp.zeros((16,), jnp.float32) for _ in range(_FDIM)]
                for c in range(4):
                    dx, dy = (c >> 1) & 1, c & 1
                    wxy = wx[dx] * wy[dy]
                    r = rowbase + c * _P
                    for dz in range(2):
                        w = wxy * wz[dz]
                        for f in range(_FDIM):
                            v = plsc.load_gather(
                                rowsl,
                                [r, jnp.full((16,), dz * _FDIM + f,
                                             jnp.int32)])
                            acc[f] = acc[f] + w * v
                cbi = g // 8
                lo = (g % 8) * 16
                for f in range(_FDIM):
                    outv[l, cbi, f, pl.ds(lo, 16)] = acc[f]

        pltpu.sync_copy(outv, feats.at[:, pl.ds(wid * (_PW // 128) + k * _NCB,
                                                _NCB), :, :])

    fire(0, bufA)

    def pair_body(j, carry):
        fire(2 * j + 1, bufB)
        combine(2 * j, bufA)
        fire(jnp.minimum(2 * j + 2, _NCHUNK - 1), bufA)
        combine(2 * j + 1, bufB)
        return carry

    lax.fori_loop(0, _NCHUNK // 2, pair_body, 0)
    for l in range(3):
        pltpu.make_async_copy(
            cbs[l].at[pl.ds(0, 4 * _P)], bufA[2][l], sem).wait()


_TBLK = (32768 // 128, 262144 // 128, 2097152 // 128)
_TSPAN = 8


def _transpose_sc_body(cbt0, cbt1, cbt2, out0, out1, out2,
                       bufA, bufB, obufA, obufB, insem, outsem):
    wid = lax.axis_index("c") * _NS + lax.axis_index("s")
    iota = lax.iota(jnp.int32, 16)
    fcol = [jnp.full((16,), f, jnp.int32) for f in range(2 * _FDIM)]

    for cbt, out, nblk in ((cbt0, out0, _TBLK[0]), (cbt1, out1, _TBLK[1]),
                           (cbt2, out2, _TBLK[2])):
        per_w = nblk // _NW
        base_blk = wid * per_w
        spans = per_w // _TSPAN

        def infire(s, buf, cbt=cbt, nblk=nblk):
            blk0 = base_blk + s * _TSPAN
            pltpu.async_copy(cbt.at[pl.ds(blk0, _TSPAN), :, :],
                             buf.at[pl.ds(0, _TSPAN), :, :], insem)
            nxt = jnp.minimum(blk0 + _TSPAN, nblk - 1)
            pltpu.async_copy(cbt.at[pl.ds(nxt, 1), :, :],
                             buf.at[pl.ds(_TSPAN, 1), :, :], insem)

        def inwait(buf, cbt=cbt):
            pltpu.make_async_copy(
                cbt.at[pl.ds(0, _TSPAN + 1), :, :], buf, insem).wait()

        def shuffle(buf, obuf):
            def blk_body(b, c2):
                for g in range(8):
                    rowi = b * 128 + g * 16 + iota
                    ip = b * 128 + g * 16 + 1 + iota
                    blkv = lax.shift_right_logical(ip, 7)
                    lanev = lax.bitwise_and(ip, 127)
                    for f in range(_FDIM):
                        v = buf[b, f, pl.ds(g * 16, 16)]
                        plsc.store_scatter(obuf, [rowi, fcol[f]], v)
                        vp = plsc.load_gather(buf, [blkv, fcol[f], lanev])
                        plsc.store_scatter(obuf, [rowi, fcol[_FDIM + f]], vp)
                return c2

            lax.fori_loop(0, _TSPAN, blk_body, 0)

        def outfire(s, obuf, out=out):
            blk0 = base_blk + s * _TSPAN
            pltpu.async_copy(obuf, out.at[pl.ds(blk0 * 128, _TSPAN * 128), :],
                             outsem)

        def outwait(obuf, out=out):
            pltpu.make_async_copy(
                obuf, out.at[pl.ds(0, _TSPAN * 128), :], outsem).wait()

        if spans == 1:
            infire(0, bufA)
            inwait(bufA)
            shuffle(bufA, obufA)
            outfire(0, obufA)
            outwait(obufA)
        else:
            infire(0, bufA)
            infire(1, bufB)
            inwait(bufA)
            shuffle(bufA, obufA)
            outfire(0, obufA)
            infire(2, bufA)
            inwait(bufB)
            shuffle(bufB, obufB)
            outfire(1, obufB)

            def pair_body(j, c2):
                infire(2 * j + 1, bufB)
                inwait(bufA)
                outwait(obufA)
                shuffle(bufA, obufA)
                outfire(2 * j, obufA)
                infire(jnp.minimum(2 * j + 2, spans - 1), bufA)
                inwait(bufB)
                outwait(obufB)
                shuffle(bufB, obufB)
                outfire(2 * j + 1, obufB)
                return c2

            lax.fori_loop(1, spans // 2, pair_body, 0)
            inwait(bufA)
            outwait(obufA)
            outwait(obufB)


_transpose_sc_cache = []


def _transpose_sc(cbt0, cbt1, cbt2):
    if not _transpose_sc_cache:
        _transpose_sc_cache.append(functools.partial(
            pl.kernel,
            out_type=(
                jax.ShapeDtypeStruct((32768, 2 * _FDIM), jnp.float32),
                jax.ShapeDtypeStruct((262144, 2 * _FDIM), jnp.float32),
                jax.ShapeDtypeStruct((2097152, 2 * _FDIM), jnp.float32),
            ),
            mesh=plsc.VectorSubcoreMesh(
                core_axis_name="c", subcore_axis_name="s",
                num_cores=_NC, num_subcores=_NS),
            scratch_types=[
                pltpu.VMEM((_TSPAN + 1, _FDIM, 128), jnp.float32),
                pltpu.VMEM((_TSPAN + 1, _FDIM, 128), jnp.float32),
                pltpu.VMEM((_TSPAN * 128, 2 * _FDIM), jnp.float32),
                pltpu.VMEM((_TSPAN * 128, 2 * _FDIM), jnp.float32),
                pltpu.SemaphoreType.DMA,
                pltpu.SemaphoreType.DMA,
            ],
            compiler_params=pltpu.CompilerParams(
                needs_layout_passes=False, use_tc_tiling_on_sc=False),
        )(_transpose_sc_body))
    return _transpose_sc_cache[0](cbt0, cbt1, cbt2)


_interp_sc_cache = []


def _interp_sc(*args):
    if not _interp_sc_cache:
        _interp_sc_cache.append(functools.partial(
            pl.kernel,
            out_type=jax.ShapeDtypeStruct((3, _N // 128, _FDIM, 128),
                                          jnp.float32),
            mesh=plsc.VectorSubcoreMesh(
                core_axis_name="c", subcore_axis_name="s",
                num_cores=_NC, num_subcores=_NS),
            scratch_types=(
                [pltpu.VMEM((3, _P), jnp.float32)]
                + [pltpu.VMEM((4 * _P,), jnp.int32)] * 3
                + [pltpu.VMEM((4 * _P, 2 * _FDIM), jnp.float32)] * 3
            ) * 2 + [
                pltpu.VMEM((3, _NCB, _FDIM, 128), jnp.float32),
                pltpu.SemaphoreType.DMA,
            ],
            compiler_params=pltpu.CompilerParams(
                needs_layout_passes=False, use_tc_tiling_on_sc=False),
        )(_interp_sc_body))
    return _interp_sc_cache[0](*args)


_BT = 4096


def _mm(w, x):
    return lax.dot_general(
        w.astype(jnp.bfloat16), x.astype(jnp.bfloat16),
        (((1,), (0,)), ((), ())), preferred_element_type=jnp.float32)


def _mlp_body(feats_ref, ptsT_ref, dT_ref, sw1t, sb1, sw2t, sb2,
              cw1ht, cw1et, cb1, cw2t, cb2, cw3t, cb3,
              colorT_ref, sigmaT_ref):
    f = feats_ref[...]
    pts = ptsT_ref[...]
    dd = dT_ref[...]
    p = pts / 3.0
    m = ((jnp.abs(p[0:1, :]) < 0.5) & (jnp.abs(p[1:2, :]) < 0.5)
         & (jnp.abs(p[2:3, :]) < 0.5))
    s = jnp.sin(dd)
    c = jnp.cos(dd)
    embs = [dd, s, c]
    for _ in range(3):
        s, c = 2.0 * s * c, 1.0 - 2.0 * s * s
        embs.append(s)
        embs.append(c)
    emb = jnp.concatenate(embs, axis=0)
    h1 = jnp.maximum(_mm(sw1t[...], f) + sb1[...], 0.0)
    h = _mm(sw2t[...], h1) + sb2[...]
    x1 = jnp.maximum(_mm(cw1ht[...], h) + _mm(cw1et[...], emb) + cb1[...], 0.0)
    x2 = jnp.maximum(_mm(cw2t[...], x1) + cb2[...], 0.0)
    logits = _mm(cw3t[...], x2) + cb3[...]
    cl = 1.0 / (1.0 + jnp.exp(-logits))
    colorT_ref[...] = jnp.where(m, cl, 0.0)
    ls = jnp.where(m, h[0:1, :], -100000.0)
    sigmaT_ref[...] = jnp.exp(ls)


def _full_spec(shape):
    return pl.BlockSpec(shape, lambda i: (0,) * len(shape))


def _mlp_call(feats, ptsT, dT, wts):
    grid = (_N // _BT,)
    in_specs = [
        pl.BlockSpec((3 * _FDIM, _BT), lambda i: (0, i)),
        pl.BlockSpec((3, _BT), lambda i: (0, i)),
        pl.BlockSpec((3, _BT), lambda i: (0, i)),
    ] + [_full_spec(w.shape) for w in wts]
    out_specs = [
        pl.BlockSpec((3, _BT), lambda i: (0, i)),
        pl.BlockSpec((1, _BT), lambda i: (0, i)),
    ]
    out_shape = [
        jax.ShapeDtypeStruct((3, _N), jnp.float32),
        jax.ShapeDtypeStruct((1, _N), jnp.float32),
    ]
    return pl.pallas_call(
        _mlp_body, grid=grid, in_specs=in_specs, out_specs=out_specs,
        out_shape=out_shape)(feats, ptsT, dT, *wts)


def kernel(pts, d, codebook0, codebook1, codebook2, sw1, sb1, sw2, sb2,
           cw1, cb1, cw2, cb2, cw3, cb3):
    ptsT = pts.T
    dT = d.T
    cbt0 = codebook0.T.reshape(_FDIM, -1, 128).transpose(1, 0, 2)
    cbt1 = codebook1.T.reshape(_FDIM, -1, 128).transpose(1, 0, 2)
    cbt2 = codebook2.T.reshape(_FDIM, -1, 128).transpose(1, 0, 2)
    cb0l, cb1l, cb2l = _transpose_sc(cbt0, cbt1, cbt2)
    feats4 = _interp_sc(ptsT, cb0l, cb1l, cb2l)
    feats = feats4.transpose(0, 2, 1, 3).reshape(3 * _FDIM, _N)
    wts = (
        sw1.T, sb1.reshape(64, 1),
        sw2.T, sb2.reshape(16, 1),
        cw1[:16].T, cw1[16:].T, cb1.reshape(64, 1),
        cw2.T, cb2.reshape(64, 1),
        cw3.T, cb3.reshape(3, 1),
    )
    colorT, sigmaT = _mlp_call(feats, ptsT, dT, wts)
    return colorT.T, sigmaT.T


# speedup vs baseline: 1.0845x; 1.0520x over previous
import functools

import jax
import jax.numpy as jnp
from jax import lax
from jax.experimental import pallas as pl
from jax.experimental.pallas import tpu as pltpu
from jax.experimental.pallas import tpu_sc as plsc

_FDIM = 8
_LODS = (32, 64, 128)
_N = 524288
_NC, _NS = 2, 16
_NW = _NC * _NS
_P = 256
_PW = _N // _NW
_NCHUNK = _PW // _P
_G = _P // 16
_JG = (4 * _P) // 128
_NCB = _P // 128


def _scaled_coords(v, res):
    p = v / 3.0 + 0.5
    s = p * float(res - 1)
    i = jnp.minimum(jnp.maximum(s, 0.0).astype(jnp.int32), res - 2)
    f = s - i.astype(jnp.float32)
    return i, f


def _interp_sc_body(ptsT, cb0, cb1, cb2, feats,
                    xyzA, idx0A, idx1A, idx2A, rows0A, rows1A, rows2A,
                    xyzB, idx0B, idx1B, idx2B, rows0B, rows1B, rows2B,
                    outv, sem, *, pw=_PW, nchunk=_NCHUNK):
    wid = lax.axis_index("c") * _NS + lax.axis_index("s")
    cbs = (cb0, cb1, cb2)
    bufA = (xyzA, (idx0A, idx1A, idx2A), (rows0A, rows1A, rows2A))
    bufB = (xyzB, (idx0B, idx1B, idx2B), (rows0B, rows1B, rows2B))
    iota = lax.iota(jnp.int32, 16)

    def fire(k, buf):
        xyz, idxs, rows = buf
        base = wid * pw + k * _P
        pltpu.sync_copy(ptsT.at[:, pl.ds(base, _P)], xyz)

        @plsc.parallel_loop(0, _G)
        def group_a(g):
            x = xyz[0, pl.ds(g * 16, 16)]
            y = xyz[1, pl.ds(g * 16, 16)]
            z = xyz[2, pl.ds(g * 16, 16)]
            for l, res in enumerate(_LODS):
                ix, _ = _scaled_coords(x, res)
                iy, _ = _scaled_coords(y, res)
                iz, _ = _scaled_coords(z, res)
                flat = (ix * res + iy) * res + iz
                for c in range(4):
                    dx, dy = (c >> 1) & 1, c & 1
                    off = dx * res * res + dy * res
                    idxs[l][pl.ds(c * _P + g * 16, 16)] = flat + off

        for l in range(3):
            for j in range(_JG):
                pltpu.async_copy(
                    cbs[l].at[idxs[l].at[pl.ds(j * 128, 128)]],
                    rows[l].at[pl.ds(j * 128, 128)], sem)

    def combine(k, buf):
        xyz, _, rows = buf
        for l, res in enumerate(_LODS):
            pltpu.make_async_copy(
                cbs[l].at[pl.ds(0, 4 * _P)], rows[l], sem).wait()

            @plsc.parallel_loop(0, _G)
            def group_b(g, l=l, res=res, rowsl=rows[l]):
                x = xyz[0, pl.ds(g * 16, 16)]
                y = xyz[1, pl.ds(g * 16, 16)]
                z = xyz[2, pl.ds(g * 16, 16)]
                rowbase = g * 16 + iota
                _, fx = _scaled_coords(x, res)
                _, fy = _scaled_coords(y, res)
                _, fz = _scaled_coords(z, res)
                wx = (1.0 - fx, fx)
                wy = (1.0 - fy, fy)
                wz = (1.0 - fz, fz)
                acc = [jnp.zeros((16,), jnp.float32) for _ in range(_FDIM)]
                for c in range(4):
                    dx, dy = (c >> 1) & 1, c & 1
                    wxy = wx[dx] * wy[dy]
                    r = rowbase + c * _P
                    for dz in range(2):
                        w = wxy * wz[dz]
                        for f in range(_FDIM):
                            v = plsc.load_gather(
                                rowsl,
                                [r, jnp.full((16,), dz * _FDIM + f,
                                             jnp.int32)])
                            acc[f] = acc[f] + w * v
                cbi = g // 8
                lo = (g % 8) * 16
                for f in range(_FDIM):
                    outv[l, cbi, f, pl.ds(lo, 16)] = acc[f]

        pltpu.sync_copy(outv, feats.at[:, pl.ds(wid * (pw // 128) + k * _NCB,
                                                _NCB), :, :])

    fire(0, bufA)

    def pair_body(j, carry):
        fire(2 * j + 1, bufB)
        combine(2 * j, bufA)
        fire(jnp.minimum(2 * j + 2, nchunk - 1), bufA)
        combine(2 * j + 1, bufB)
        return carry

    lax.fori_loop(0, nchunk // 2, pair_body, 0)
    for l in range(3):
        pltpu.make_async_copy(
            cbs[l].at[pl.ds(0, 4 * _P)], bufA[2][l], sem).wait()


_TBLK = (32768 // 128, 262144 // 128, 2097152 // 128)
_TSPAN = 8


def _transpose_sc_body(cbt0, cbt1, cbt2, out0, out1, out2,
                       bufA, bufB, obufA, obufB, insem, outsem):
    wid = lax.axis_index("c") * _NS + lax.axis_index("s")
    iota = lax.iota(jnp.int32, 16)
    fcol = [jnp.full((16,), f, jnp.int32) for f in range(2 * _FDIM)]

    for cbt, out, nblk in ((cbt0, out0, _TBLK[0]), (cbt1, out1, _TBLK[1]),
                           (cbt2, out2, _TBLK[2])):
        per_w = nblk // _NW
        base_blk = wid * per_w
        spans = per_w // _TSPAN

        def infire(s, buf, cbt=cbt, nblk=nblk):
            blk0 = base_blk + s * _TSPAN
            pltpu.async_copy(cbt.at[pl.ds(blk0, _TSPAN), :, :],
                             buf.at[pl.ds(0, _TSPAN), :, :], insem)
            nxt = jnp.minimum(blk0 + _TSPAN, nblk - 1)
            pltpu.async_copy(cbt.at[pl.ds(nxt, 1), :, :],
                             buf.at[pl.ds(_TSPAN, 1), :, :], insem)

        def inwait(buf, cbt=cbt):
            pltpu.make_async_copy(
                cbt.at[pl.ds(0, _TSPAN + 1), :, :], buf, insem).wait()

        def shuffle(buf, obuf):
            def blk_body(b, c2):
                for g in range(8):
                    rowi = b * 128 + g * 16 + iota
                    ip = b * 128 + g * 16 + 1 + iota
                    blkv = lax.shift_right_logical(ip, 7)
                    lanev = lax.bitwise_and(ip, 127)
                    for f in range(_FDIM):
                        v = buf[b, f, pl.ds(g * 16, 16)]
                        plsc.store_scatter(obuf, [rowi, fcol[f]], v)
                        vp = plsc.load_gather(buf, [blkv, fcol[f], lanev])
                        plsc.store_scatter(obuf, [rowi, fcol[_FDIM + f]], vp)
                return c2

            lax.fori_loop(0, _TSPAN, blk_body, 0)

        def outfire(s, obuf, out=out):
            blk0 = base_blk + s * _TSPAN
            pltpu.async_copy(obuf, out.at[pl.ds(blk0 * 128, _TSPAN * 128), :],
                             outsem)

        def outwait(obuf, out=out):
            pltpu.make_async_copy(
                obuf, out.at[pl.ds(0, _TSPAN * 128), :], outsem).wait()

        if spans == 1:
            infire(0, bufA)
            inwait(bufA)
            shuffle(bufA, obufA)
            outfire(0, obufA)
            outwait(obufA)
        else:
            infire(0, bufA)
            infire(1, bufB)
            inwait(bufA)
            shuffle(bufA, obufA)
            outfire(0, obufA)
            infire(2, bufA)
            inwait(bufB)
            shuffle(bufB, obufB)
            outfire(1, obufB)

            def pair_body(j, c2):
                infire(2 * j + 1, bufB)
                inwait(bufA)
                outwait(obufA)
                shuffle(bufA, obufA)
                outfire(2 * j, obufA)
                infire(jnp.minimum(2 * j + 2, spans - 1), bufA)
                inwait(bufB)
                outwait(obufB)
                shuffle(bufB, obufB)
                outfire(2 * j + 1, obufB)
                return c2

            lax.fori_loop(1, spans // 2, pair_body, 0)
            inwait(bufA)
            outwait(obufA)
            outwait(obufB)


_transpose_sc_cache = []


def _transpose_sc(cbt0, cbt1, cbt2):
    if not _transpose_sc_cache:
        _transpose_sc_cache.append(functools.partial(
            pl.kernel,
            out_type=(
                jax.ShapeDtypeStruct((32768, 2 * _FDIM), jnp.float32),
                jax.ShapeDtypeStruct((262144, 2 * _FDIM), jnp.float32),
                jax.ShapeDtypeStruct((2097152, 2 * _FDIM), jnp.float32),
            ),
            mesh=plsc.VectorSubcoreMesh(
                core_axis_name="c", subcore_axis_name="s",
                num_cores=_NC, num_subcores=_NS),
            scratch_types=[
                pltpu.VMEM((_TSPAN + 1, _FDIM, 128), jnp.float32),
                pltpu.VMEM((_TSPAN + 1, _FDIM, 128), jnp.float32),
                pltpu.VMEM((_TSPAN * 128, 2 * _FDIM), jnp.float32),
                pltpu.VMEM((_TSPAN * 128, 2 * _FDIM), jnp.float32),
                pltpu.SemaphoreType.DMA,
                pltpu.SemaphoreType.DMA,
            ],
            compiler_params=pltpu.CompilerParams(
                needs_layout_passes=False, use_tc_tiling_on_sc=False),
        )(_transpose_sc_body))
    return _transpose_sc_cache[0](cbt0, cbt1, cbt2)


_interp_sc_cache = {}


def _interp_sc(ptsT_part, cb0l, cb1l, cb2l):
    n = ptsT_part.shape[1]
    if n not in _interp_sc_cache:
        pw = n // _NW
        _interp_sc_cache[n] = functools.partial(
            pl.kernel,
            out_type=jax.ShapeDtypeStruct((3, n // 128, _FDIM, 128),
                                          jnp.float32),
            mesh=plsc.VectorSubcoreMesh(
                core_axis_name="c", subcore_axis_name="s",
                num_cores=_NC, num_subcores=_NS),
            scratch_types=(
                [pltpu.VMEM((3, _P), jnp.float32)]
                + [pltpu.VMEM((4 * _P,), jnp.int32)] * 3
                + [pltpu.VMEM((4 * _P, 2 * _FDIM), jnp.float32)] * 3
            ) * 2 + [
                pltpu.VMEM((3, _NCB, _FDIM, 128), jnp.float32),
                pltpu.SemaphoreType.DMA,
            ],
            compiler_params=pltpu.CompilerParams(
                needs_layout_passes=False, use_tc_tiling_on_sc=False),
        )(functools.partial(_interp_sc_body, pw=pw, nchunk=pw // _P))
    return _interp_sc_cache[n](ptsT_part, cb0l, cb1l, cb2l)


_BT = 4096


def _mm(w, x):
    return lax.dot_general(
        w.astype(jnp.bfloat16), x.astype(jnp.bfloat16),
        (((1,), (0,)), ((), ())), preferred_element_type=jnp.float32)


def _mlp_body(feats_ref, ptsT_ref, dT_ref, sw1t, sb1, sw2t, sb2,
              cw1ht, cw1et, cb1, cw2t, cb2, cw3t, cb3,
              colorT_ref, sigmaT_ref):
    f = feats_ref[...]
    pts = ptsT_ref[...]
    dd = dT_ref[...]
    p = pts / 3.0
    m = ((jnp.abs(p[0:1, :]) < 0.5) & (jnp.abs(p[1:2, :]) < 0.5)
         & (jnp.abs(p[2:3, :]) < 0.5))
    s = jnp.sin(dd)
    c = jnp.cos(dd)
    embs = [dd, s, c]
    for _ in range(3):
        s, c = 2.0 * s * c, 1.0 - 2.0 * s * s
        embs.append(s)
        embs.append(c)
    emb = jnp.concatenate(embs, axis=0)
    h1 = jnp.maximum(_mm(sw1t[...], f) + sb1[...], 0.0)
    h = _mm(sw2t[...], h1) + sb2[...]
    x1 = jnp.maximum(_mm(cw1ht[...], h) + _mm(cw1et[...], emb) + cb1[...], 0.0)
    x2 = jnp.maximum(_mm(cw2t[...], x1) + cb2[...], 0.0)
    logits = _mm(cw3t[...], x2) + cb3[...]
    cl = 1.0 / (1.0 + jnp.exp(-logits))
    colorT_ref[...] = jnp.where(m, cl, 0.0)
    ls = jnp.where(m, h[0:1, :], -100000.0)
    sigmaT_ref[...] = jnp.exp(ls)


def _full_spec(shape):
    return pl.BlockSpec(shape, lambda i: (0,) * len(shape))


def _mlp_call(feats, ptsT, dT, wts):
    n = feats.shape[1]
    grid = (n // _BT,)
    in_specs = [
        pl.BlockSpec((3 * _FDIM, _BT), lambda i: (0, i)),
        pl.BlockSpec((3, _BT), lambda i: (0, i)),
        pl.BlockSpec((3, _BT), lambda i: (0, i)),
    ] + [_full_spec(w.shape) for w in wts]
    out_specs = [
        pl.BlockSpec((3, _BT), lambda i: (0, i)),
        pl.BlockSpec((1, _BT), lambda i: (0, i)),
    ]
    out_shape = [
        jax.ShapeDtypeStruct((3, n), jnp.float32),
        jax.ShapeDtypeStruct((1, n), jnp.float32),
    ]
    return pl.pallas_call(
        _mlp_body, grid=grid, in_specs=in_specs, out_specs=out_specs,
        out_shape=out_shape)(feats, ptsT, dT, *wts)


def kernel(pts, d, codebook0, codebook1, codebook2, sw1, sb1, sw2, sb2,
           cw1, cb1, cw2, cb2, cw3, cb3):
    ptsT = pts.T
    dT = d.T
    cbt0 = codebook0.T.reshape(_FDIM, -1, 128).transpose(1, 0, 2)
    cbt1 = codebook1.T.reshape(_FDIM, -1, 128).transpose(1, 0, 2)
    cbt2 = codebook2.T.reshape(_FDIM, -1, 128).transpose(1, 0, 2)
    cb0l, cb1l, cb2l = _transpose_sc(cbt0, cbt1, cbt2)
    wts = (
        sw1.T, sb1.reshape(64, 1),
        sw2.T, sb2.reshape(16, 1),
        cw1[:16].T, cw1[16:].T, cb1.reshape(64, 1),
        cw2.T, cb2.reshape(64, 1),
        cw3.T, cb3.reshape(3, 1),
    )
    h = _N // 2
    colors, sigmas = [], []
    for part in range(2):
        ptsTp = lax.slice(ptsT, (0, part * h), (3, (part + 1) * h))
        dTp = lax.slice(dT, (0, part * h), (3, (part + 1) * h))
        feats4 = _interp_sc(ptsTp, cb0l, cb1l, cb2l)
        feats = feats4.transpose(0, 2, 1, 3).reshape(3 * _FDIM, h)
        colorTp, sigmaTp = _mlp_call(feats, ptsTp, dTp, wts)
        colors.append(colorTp)
        sigmas.append(sigmaTp)
    colorT = jnp.concatenate(colors, axis=1)
    sigmaT = jnp.concatenate(sigmas, axis=1)
    return colorT.T, sigmaT.T
